# baseline (speedup 1.0000x reference)
import functools

import jax
import jax.numpy as jnp
from jax.experimental import pallas as pl
from jax.experimental.pallas import tpu as pltpu


def _ffn_body(x_ref, wgu_ref, ws_ref, w2p_ref, sgw_ref,
              out_ref, act_ref, gtok_ref, *, j_main, bi):
    j = pl.program_id(1)
    x = x_ref[...]
    dn = (((1,), (1,)), ((), ()))

    @pl.when(j == 0)
    def _init():
        act_ref[1] = jnp.zeros_like(act_ref[1])
        glogit = jax.lax.dot_general(
            x.astype(jnp.float32), sgw_ref[...].astype(jnp.float32),
            dn, preferred_element_type=jnp.float32)
        gtok_ref[...] = jax.nn.sigmoid(glogit)

    p = jax.lax.dot_general(x, wgu_ref[0], dn,
                            preferred_element_type=jnp.float32)
    s = jax.lax.dot_general(x, ws_ref[0], dn,
                            preferred_element_type=jnp.float32)
    acc = jax.lax.dot_general(act_ref[(j + 1) % 2], w2p_ref[0], dn,
                              preferred_element_type=jnp.float32)

    @pl.when(j == 0)
    def _set():
        out_ref[...] = jnp.zeros_like(out_ref)

    @pl.when(j > 0)
    def _accum():
        out_ref[...] += acc

    g = p[:, :bi]
    u = p[:, bi:]
    gtok = gtok_ref[...]
    act_main = (g * jax.nn.sigmoid(g)) * u
    act_shared = (s * jax.nn.sigmoid(s)) * gtok
    act_ref[j % 2] = jnp.concatenate(
        [act_main, act_shared], axis=1).astype(jnp.bfloat16)


def kernel(hidden_states, w13, w2, gate, shared_w1, shared_w2, shared_gate_w):
    del gate
    bsz, seq_len, hidden = hidden_states.shape
    inter = shared_w1.shape[0]
    n_tokens = bsz * seq_len

    bt = min(1024, n_tokens)
    bi = min(512, inter)
    assert n_tokens % bt == 0 and inter % bi == 0
    n_t = n_tokens // bt
    j_main = inter // bi

    x = hidden_states.reshape(n_tokens, hidden).astype(jnp.bfloat16)
    wgu = (w13.reshape(2, j_main, bi, hidden)
              .transpose(1, 0, 2, 3)
              .reshape(j_main, 2 * bi, hidden)
              .astype(jnp.bfloat16))
    wst = shared_w1.reshape(j_main, bi, hidden).astype(jnp.bfloat16)
    w2p = (jnp.concatenate(
               [w2.reshape(hidden, j_main, 1, bi),
                shared_w2.reshape(hidden, j_main, 1, bi)], axis=2)
              .transpose(1, 0, 2, 3)
              .reshape(j_main, hidden, 2 * bi)
              .astype(jnp.bfloat16))
    sgwb = shared_gate_w.astype(jnp.bfloat16)

    cm = j_main - 1

    grid_spec = pltpu.PrefetchScalarGridSpec(
        num_scalar_prefetch=0,
        grid=(n_t, j_main + 1),
        in_specs=[
            pl.BlockSpec((bt, hidden), lambda t, j: (t, 0)),
            pl.BlockSpec((1, 2 * bi, hidden),
                         lambda t, j: (jnp.minimum(j, cm), 0, 0)),
            pl.BlockSpec((1, bi, hidden),
                         lambda t, j: (jnp.minimum(j, cm), 0, 0)),
            pl.BlockSpec((1, hidden, 2 * bi),
                         lambda t, j: (jnp.clip(j - 1, 0, cm), 0, 0)),
            pl.BlockSpec((1, hidden), lambda t, j: (0, 0)),
        ],
        out_specs=pl.BlockSpec((bt, hidden), lambda t, j: (t, 0)),
        scratch_shapes=[
            pltpu.VMEM((2, bt, 2 * bi), jnp.bfloat16),
            pltpu.VMEM((bt, 1), jnp.float32),
        ],
    )

    out = pl.pallas_call(
        functools.partial(_ffn_body, j_main=j_main, bi=bi),
        grid_spec=grid_spec,
        out_shape=jax.ShapeDtypeStruct((n_tokens, hidden), jnp.float32),
        compiler_params=pltpu.CompilerParams(
            dimension_semantics=("parallel", "arbitrary"),
            vmem_limit_bytes=63 * 1024 * 1024,
        ),
    )(x, wgu, wst, w2p, sgwb)

    return out.reshape(bsz, seq_len, hidden)


# speedup vs baseline: 1.3024x; 1.3024x over previous
import functools

import jax
import jax.numpy as jnp
from jax.experimental import pallas as pl
from jax.experimental.pallas import tpu as pltpu


def _ffn_body(x_ref, wg_ref, wu_ref, ws_ref, w2_ref, sw2_ref, sgw_ref,
              out_ref, actm_ref, acts_ref, gtok_ref, *, j_main):
    j = pl.program_id(1)
    x = x_ref[...]
    dn = (((1,), (1,)), ((), ()))

    @pl.when(j == 0)
    def _init():
        actm_ref[1] = jnp.zeros_like(actm_ref[1])
        acts_ref[1] = jnp.zeros_like(acts_ref[1])
        glogit = jax.lax.dot_general(
            x.astype(jnp.float32), sgw_ref[...].astype(jnp.float32),
            dn, preferred_element_type=jnp.float32)
        gtok_ref[...] = jax.nn.sigmoid(glogit)

    g = jax.lax.dot_general(x, wg_ref[0], dn,
                            preferred_element_type=jnp.float32)
    u = jax.lax.dot_general(x, wu_ref[0], dn,
                            preferred_element_type=jnp.float32)
    s = jax.lax.dot_general(x, ws_ref[...], dn,
                            preferred_element_type=jnp.float32)
    acc1 = jax.lax.dot_general(actm_ref[(j + 1) % 2], w2_ref[...], dn,
                               preferred_element_type=jnp.float32)
    acc2 = jax.lax.dot_general(acts_ref[(j + 1) % 2], sw2_ref[...], dn,
                               preferred_element_type=jnp.float32)

    @pl.when(j == 0)
    def _set():
        out_ref[...] = jnp.zeros_like(out_ref)

    @pl.when(j > 0)
    def _accum():
        out_ref[...] += acc1 + acc2

    gtok = gtok_ref[...]
    actm_ref[j % 2] = ((g * jax.nn.sigmoid(g)) * u).astype(jnp.bfloat16)
    acts_ref[j % 2] = ((s * jax.nn.sigmoid(s)) * gtok).astype(jnp.bfloat16)


def kernel(hidden_states, w13, w2, gate, shared_w1, shared_w2, shared_gate_w):
    del gate
    bsz, seq_len, hidden = hidden_states.shape
    inter = shared_w1.shape[0]
    n_tokens = bsz * seq_len

    bt = min(1024, n_tokens)
    bi = min(256, inter)
    assert n_tokens % bt == 0 and inter % bi == 0
    n_t = n_tokens // bt
    j_main = inter // bi

    x = hidden_states.reshape(n_tokens, hidden).astype(jnp.bfloat16)
    w13r = w13.reshape(2, inter, hidden).astype(jnp.bfloat16)
    sw1b = shared_w1.astype(jnp.bfloat16)
    w2b = w2.astype(jnp.bfloat16)
    sw2b = shared_w2.astype(jnp.bfloat16)
    sgwb = shared_gate_w.astype(jnp.bfloat16)

    cm = j_main - 1

    grid_spec = pltpu.PrefetchScalarGridSpec(
        num_scalar_prefetch=0,
        grid=(n_t, j_main + 1),
        in_specs=[
            pl.BlockSpec((bt, hidden), lambda t, j: (t, 0)),
            pl.BlockSpec((1, bi, hidden),
                         lambda t, j: (0, jnp.minimum(j, cm), 0)),
            pl.BlockSpec((1, bi, hidden),
                         lambda t, j: (1, jnp.minimum(j, cm), 0)),
            pl.BlockSpec((bi, hidden),
                         lambda t, j: (jnp.minimum(j, cm), 0)),
            pl.BlockSpec((hidden, bi),
                         lambda t, j: (0, jnp.clip(j - 1, 0, cm))),
            pl.BlockSpec((hidden, bi),
                         lambda t, j: (0, jnp.clip(j - 1, 0, cm))),
            pl.BlockSpec((1, hidden), lambda t, j: (0, 0)),
        ],
        out_specs=pl.BlockSpec((bt, hidden), lambda t, j: (t, 0)),
        scratch_shapes=[
            pltpu.VMEM((2, bt, bi), jnp.bfloat16),
            pltpu.VMEM((2, bt, bi), jnp.bfloat16),
            pltpu.VMEM((bt, 1), jnp.float32),
        ],
    )

    out = pl.pallas_call(
        functools.partial(_ffn_body, j_main=j_main),
        grid_spec=grid_spec,
        out_shape=jax.ShapeDtypeStruct((n_tokens, hidden), jnp.float32),
        compiler_params=pltpu.CompilerParams(
            dimension_semantics=("parallel", "arbitrary"),
            vmem_limit_bytes=63 * 1024 * 1024,
        ),
    )(x, w13r, w13r, sw1b, w2b, sw2b, sgwb)

    return out.reshape(bsz, seq_len, hidden)
